# baseline (speedup 1.0000x reference)
import functools

import jax
import jax.numpy as jnp
from jax import lax
from jax.experimental import pallas as pl
from jax.experimental.pallas import tpu as pltpu
from jax.experimental.pallas import tpu_sc as plsc

B = 4096
F = 26
V = 100000
D = 32
NUM = 13
H1 = 512
H2 = 256
OUT = 100
EPS = 1e-5

NC = 2
NS = 16
NW = NC * NS
CHUNK = 128
NBLK = B // CHUNK
NCHUNK = F * NBLK // NW


def _gather_call(tables, idx):
    mesh = plsc.VectorSubcoreMesh(
        core_axis_name="c", subcore_axis_name="s", num_cores=NC, num_subcores=NS
    )

    @functools.partial(
        pl.kernel,
        mesh=mesh,
        compiler_params=pltpu.CompilerParams(use_tc_tiling_on_sc=False),
        out_type=jax.ShapeDtypeStruct((F, NBLK, CHUNK, D), jnp.float32),
        scratch_types=[
            pltpu.VMEM((NCHUNK, CHUNK), jnp.int32),
            pltpu.VMEM((NCHUNK, CHUNK, D), jnp.float32),
            pltpu.SemaphoreType.DMA,
        ],
    )
    def gather_k(tab_hbm, idx_hbm, out_hbm, idx_v, rows_v, sem):
        wid = lax.axis_index("s") * NC + lax.axis_index("c")
        c0 = wid * NCHUNK
        pltpu.sync_copy(idx_hbm.at[wid], idx_v)
        copies = [
            pltpu.async_copy(
                tab_hbm.at[(c0 + j) // NBLK].at[idx_v.at[j]], rows_v.at[j], sem
            )
            for j in range(NCHUNK)
        ]
        for j, cp in enumerate(copies):
            cp.wait()
            c = c0 + j
            pltpu.sync_copy(rows_v.at[j], out_hbm.at[c // NBLK].at[c % NBLK])

    return gather_k(tables, idx)


def _bn(x, g, b):
    m = jnp.mean(x, axis=0, keepdims=True)
    v = jnp.mean((x - m) * (x - m), axis=0, keepdims=True)
    return g * (x - m) / jnp.sqrt(v + EPS) + b


def _mlp_body(emb_ref, xn_ref, bng_ref, bnb_ref, w1e_ref, w1n_ref, b1_ref,
              g1_ref, be1_ref, w2_ref, b2_ref, g2_ref, be2_ref, w3_ref,
              b3_ref, out_ref):
    dn = (((1,), (1,)), ((), ()))
    xnb = _bn(xn_ref[:], bng_ref[:], bnb_ref[:])
    h = lax.dot_general(emb_ref[:], w1e_ref[:], dn,
                        preferred_element_type=jnp.float32)
    h = h + lax.dot_general(xnb, w1n_ref[:], dn,
                            preferred_element_type=jnp.float32)
    h = jnp.maximum(h + b1_ref[:], 0.0)
    h = _bn(h, g1_ref[:], be1_ref[:])
    h = lax.dot_general(h, w2_ref[:], dn, preferred_element_type=jnp.float32)
    h = jnp.maximum(h + b2_ref[:], 0.0)
    h = _bn(h, g2_ref[:], be2_ref[:])
    out_ref[:] = (
        lax.dot_general(h, w3_ref[:], dn, preferred_element_type=jnp.float32)
        + b3_ref[:]
    )


def kernel(x_categorical, x_numerical, emb_tables, bn_num_g, bn_num_b,
           W1, b1, g1, be1, W2, b2, g2, be2, W3, b3):
    idx = x_categorical.astype(jnp.int32).T.reshape(NW, NCHUNK, CHUNK)
    emb4 = _gather_call(emb_tables, idx)
    emb = emb4.transpose(1, 2, 0, 3).reshape(B, F * D)

    out = pl.pallas_call(
        _mlp_body,
        out_shape=jax.ShapeDtypeStruct((B, OUT), jnp.float32),
    )(
        emb,
        x_numerical,
        bn_num_g[None, :],
        bn_num_b[None, :],
        W1[:, : F * D],
        W1[:, F * D:],
        b1[None, :],
        g1[None, :],
        be1[None, :],
        W2,
        b2[None, :],
        g2[None, :],
        be2[None, :],
        W3,
        b3[None, :],
    )
    return out


# speedup vs baseline: 1.7762x; 1.7762x over previous
import functools

import jax
import jax.numpy as jnp
from jax import lax
from jax.experimental import pallas as pl
from jax.experimental.pallas import tpu as pltpu
from jax.experimental.pallas import tpu_sc as plsc

B = 4096
F = 26
V = 100000
D = 32
NUM = 13
H1 = 512
H2 = 256
OUT = 100
EPS = 1e-5

NC = 2
NS = 16
NW = NC * NS
CHUNK = 128
NBLK = B // CHUNK
NCHUNK = F * NBLK // NW
RPW = F * D // NW


def _gather_call(tabT, idxT):
    mesh = plsc.VectorSubcoreMesh(
        core_axis_name="c", subcore_axis_name="s", num_cores=NC, num_subcores=NS
    )

    @functools.partial(
        pl.kernel,
        mesh=mesh,
        compiler_params=pltpu.CompilerParams(
            use_tc_tiling_on_sc=False, needs_layout_passes=False),
        out_type=jax.ShapeDtypeStruct((F, D, B), jnp.float32),
        scratch_types=[
            pltpu.VMEM((V,), jnp.float32),
            pltpu.VMEM((B,), jnp.int32),
            pltpu.VMEM((B,), jnp.float32),
        ],
    )
    def gather_k(tab_hbm, idx_hbm, out_hbm, row_v, idx_v, stage_v):
        wid = lax.axis_index("s") * NC + lax.axis_index("c")
        r0 = wid * RPW

        def row(i, carry):
            r = r0 + i
            f = r // D
            d = r % D
            pltpu.sync_copy(idx_hbm.at[f], idx_v)
            pltpu.sync_copy(tab_hbm.at[f].at[d], row_v)

            def grp(g, c2):
                s = pl.ds(g * 16, 16)
                stage_v[s] = plsc.load_gather(row_v, [idx_v[s]])
                return c2

            lax.fori_loop(0, B // 16, grp, 0)
            pltpu.sync_copy(stage_v, out_hbm.at[f].at[d])
            return carry

        lax.fori_loop(0, RPW, row, 0)

    return gather_k(tabT, idxT)


def _bn(x, g, b):
    m = jnp.mean(x, axis=0, keepdims=True)
    v = jnp.mean((x - m) * (x - m), axis=0, keepdims=True)
    return g * (x - m) / jnp.sqrt(v + EPS) + b


def _mlp_body(emb_ref, xn_ref, bng_ref, bnb_ref, w1e_ref, w1n_ref, b1_ref,
              g1_ref, be1_ref, w2_ref, b2_ref, g2_ref, be2_ref, w3_ref,
              b3_ref, out_ref):
    dn = (((1,), (1,)), ((), ()))
    xnb = _bn(xn_ref[:], bng_ref[:], bnb_ref[:])
    h = lax.dot_general(emb_ref[:], w1e_ref[:], dn,
                        preferred_element_type=jnp.float32)
    h = h + lax.dot_general(xnb, w1n_ref[:], dn,
                            preferred_element_type=jnp.float32)
    h = jnp.maximum(h + b1_ref[:], 0.0)
    h = _bn(h, g1_ref[:], be1_ref[:])
    h = lax.dot_general(h, w2_ref[:], dn, preferred_element_type=jnp.float32)
    h = jnp.maximum(h + b2_ref[:], 0.0)
    h = _bn(h, g2_ref[:], be2_ref[:])
    out_ref[:] = (
        lax.dot_general(h, w3_ref[:], dn, preferred_element_type=jnp.float32)
        + b3_ref[:]
    )


def kernel(x_categorical, x_numerical, emb_tables, bn_num_g, bn_num_b,
           W1, b1, g1, be1, W2, b2, g2, be2, W3, b3):
    idxT = x_categorical.astype(jnp.int32).T
    emb4 = _gather_call(emb_tables.transpose(0, 2, 1), idxT)
    emb = emb4.transpose(2, 0, 1).reshape(B, F * D)

    out = pl.pallas_call(
        _mlp_body,
        out_shape=jax.ShapeDtypeStruct((B, OUT), jnp.float32),
    )(
        emb,
        x_numerical,
        bn_num_g[None, :],
        bn_num_b[None, :],
        W1[:, : F * D],
        W1[:, F * D:],
        b1[None, :],
        g1[None, :],
        be1[None, :],
        W2,
        b2[None, :],
        g2[None, :],
        be2[None, :],
        W3,
        b3[None, :],
    )
    return out


# speedup vs baseline: 1.8353x; 1.0333x over previous
import functools

import jax
import jax.numpy as jnp
from jax import lax
from jax.experimental import pallas as pl
from jax.experimental.pallas import tpu as pltpu
from jax.experimental.pallas import tpu_sc as plsc

B = 4096
F = 26
V = 100000
D = 32
NUM = 13
H1 = 512
H2 = 256
OUT = 100
EPS = 1e-5

NC = 2
NS = 16
NW = NC * NS
CHUNK = 128
NBLK = B // CHUNK
NCHUNK = F * NBLK // NW
RPW = F * D // NW


def _gather_call(tabT, idxT):
    mesh = plsc.VectorSubcoreMesh(
        core_axis_name="c", subcore_axis_name="s", num_cores=NC, num_subcores=NS
    )

    @functools.partial(
        pl.kernel,
        mesh=mesh,
        compiler_params=pltpu.CompilerParams(
            use_tc_tiling_on_sc=False, needs_layout_passes=False),
        out_type=jax.ShapeDtypeStruct((F, D, B), jnp.float32),
        scratch_types=[
            pltpu.VMEM((V,), jnp.float32),
            pltpu.VMEM((B,), jnp.int32),
            pltpu.VMEM((2, B), jnp.float32),
            [pltpu.SemaphoreType.DMA] * 2,
        ],
    )
    def gather_k(tab_hbm, idx_hbm, out_hbm, row_v, idx_v, stage_v, osems):
        wid = lax.axis_index("s") * NC + lax.axis_index("c")
        r0 = wid * RPW
        pltpu.sync_copy(idx_hbm.at[r0 // D], idx_v)

        def do_row(r, ob, prev_f):
            f = r // D
            d = r % D

            @pl.when(f != prev_f)
            def _():
                pltpu.sync_copy(idx_hbm.at[f], idx_v)

            pltpu.sync_copy(tab_hbm.at[f].at[d], row_v)

            def grp(g, c2):
                s = pl.ds(g * 16, 16)
                stage_v[ob, s] = plsc.load_gather(row_v, [idx_v[s]])
                return c2

            lax.fori_loop(0, B // 16, grp, 0)
            pltpu.async_copy(stage_v.at[ob], out_hbm.at[f].at[d], osems[ob])
            return f

        def pair(p, prev_f):
            r = r0 + p * 2
            prev_f = do_row(r, 0, prev_f)

            @pl.when(p > 0)
            def _():
                pltpu.make_async_copy(
                    stage_v.at[1], out_hbm.at[0].at[0], osems[1]).wait()

            prev_f = do_row(r + 1, 1, prev_f)
            pltpu.make_async_copy(
                stage_v.at[0], out_hbm.at[0].at[0], osems[0]).wait()
            return prev_f

        lax.fori_loop(0, RPW // 2, pair, r0 // D)
        pltpu.make_async_copy(stage_v.at[1], out_hbm.at[0].at[0],
                              osems[1]).wait()

    return gather_k(tabT, idxT)


def _bn(x, g, b):
    m = jnp.mean(x, axis=0, keepdims=True)
    v = jnp.mean((x - m) * (x - m), axis=0, keepdims=True)
    return g * (x - m) / jnp.sqrt(v + EPS) + b


def _mlp_body(emb_ref, xn_ref, bng_ref, bnb_ref, w1e_ref, w1n_ref, b1_ref,
              g1_ref, be1_ref, w2_ref, b2_ref, g2_ref, be2_ref, w3_ref,
              b3_ref, out_ref):
    dn = (((1,), (1,)), ((), ()))
    xnb = _bn(xn_ref[:], bng_ref[:], bnb_ref[:])
    h = lax.dot_general(emb_ref[:], w1e_ref[:], dn,
                        preferred_element_type=jnp.float32)
    h = h + lax.dot_general(xnb, w1n_ref[:], dn,
                            preferred_element_type=jnp.float32)
    h = jnp.maximum(h + b1_ref[:], 0.0)
    h = _bn(h, g1_ref[:], be1_ref[:])
    h = lax.dot_general(h, w2_ref[:], dn, preferred_element_type=jnp.float32)
    h = jnp.maximum(h + b2_ref[:], 0.0)
    h = _bn(h, g2_ref[:], be2_ref[:])
    out_ref[:] = (
        lax.dot_general(h, w3_ref[:], dn, preferred_element_type=jnp.float32)
        + b3_ref[:]
    )


def kernel(x_categorical, x_numerical, emb_tables, bn_num_g, bn_num_b,
           W1, b1, g1, be1, W2, b2, g2, be2, W3, b3):
    idxT = x_categorical.astype(jnp.int32).T
    emb4 = _gather_call(emb_tables.transpose(0, 2, 1), idxT)
    emb = emb4.transpose(2, 0, 1).reshape(B, F * D)

    out = pl.pallas_call(
        _mlp_body,
        out_shape=jax.ShapeDtypeStruct((B, OUT), jnp.float32),
    )(
        emb,
        x_numerical,
        bn_num_g[None, :],
        bn_num_b[None, :],
        W1[:, : F * D],
        W1[:, F * D:],
        b1[None, :],
        g1[None, :],
        be1[None, :],
        W2,
        b2[None, :],
        g2[None, :],
        be2[None, :],
        W3,
        b3[None, :],
    )
    return out
